# baseline (speedup 1.0000x reference)
import functools

import jax
import jax.numpy as jnp
from jax import lax
from jax.experimental import pallas as pl
from jax.experimental.pallas import tpu as pltpu
from jax.experimental.pallas import tpu_sc as plsc

N = 10000
N_PAD = 10240
E = 320000
NC, NS = 2, 16
NW = NC * NS
E_PER_W = E // NW
D = 256

BLK = 1280
N_BLKS = E // BLK
STAG = 3584
FLUSH = 2048
FINAL_FLUSH = 2176
LIST_CAP = E + 4096
GRP = 64
ROWS_PER_TILE = 320
TRASH = ROWS_PER_TILE
ACC_ROWS = ROWS_PER_TILE + 8


def _mesh():
    return plsc.VectorSubcoreMesh(core_axis_name="c", subcore_axis_name="s",
                                  num_cores=NC, num_subcores=NS)


def _wid():
    return lax.axis_index("s") * NC + lax.axis_index("c")


@functools.cache
def _get_route_kernel():
    return pl.kernel(
        _route_body,
        out_type=(
            jax.ShapeDtypeStruct((NW, LIST_CAP), jnp.int32),
            jax.ShapeDtypeStruct((NW, LIST_CAP), jnp.int32),
            jax.ShapeDtypeStruct((NW, 16), jnp.int32),
            jax.ShapeDtypeStruct((NW, N_PAD), jnp.float32),
        ),
        mesh=_mesh(),
        scratch_types=[
            pltpu.VMEM((BLK,), jnp.int32),
            pltpu.VMEM((BLK,), jnp.int32),
            pltpu.VMEM((BLK,), jnp.int32),
            pltpu.VMEM((BLK,), jnp.int32),
            pltpu.VMEM((STAG,), jnp.int32),
            pltpu.VMEM((STAG,), jnp.int32),
            pltpu.VMEM((16,), jnp.int32),
            pltpu.VMEM((N_PAD,), jnp.float32),
            pltpu.SemaphoreType.DMA,
            pltpu.SemaphoreType.DMA,
        ],
        compiler_params=pltpu.CompilerParams(needs_layout_passes=False),
    )


def _route_body(src_hbm, dst_hbm, slist_hbm, llist_hbm, cnt_hbm, deg_hbm,
                sbuf0, dbuf0, sbuf1, dbuf1, stag_s, stag_l, cbuf, degv,
                semb0, semb1):
    w = _wid()
    w_vec = jnp.full((16,), 0, jnp.int32) + w
    sbuf = (sbuf0, sbuf1)
    dbuf = (dbuf0, dbuf1)
    semb = (semb0, semb1)
    ones16 = jnp.ones((16,), jnp.float32)

    pltpu.async_copy(src_hbm.at[pl.ds(0, BLK)], sbuf0, semb0)
    pltpu.async_copy(dst_hbm.at[pl.ds(0, BLK)], dbuf0, semb0)

    def _zero(i, carry):
        degv[pl.ds(i * 16, 16)] = jnp.zeros((16,), jnp.float32)
        return carry

    lax.fori_loop(0, N_PAD // 16, _zero, 0)

    def _pairblk(p, carry):
        for b in range(2):
            j = p * 2 + b
            pltpu.make_async_copy(src_hbm.at[pl.ds(0, BLK)], sbuf[b],
                                  semb[b]).wait()
            pltpu.make_async_copy(dst_hbm.at[pl.ds(0, BLK)], dbuf[b],
                                  semb[b]).wait()

            @pl.when(j + 1 < N_BLKS)
            def _prefetch():
                nb = 1 - b
                off = (j + 1) * BLK
                pltpu.async_copy(src_hbm.at[pl.ds(off, BLK)], sbuf[nb],
                                 semb[nb])
                pltpu.async_copy(dst_hbm.at[pl.ds(off, BLK)], dbuf[nb],
                                 semb[nb])

            n, flushed = carry
            for t in range(BLK // 16):
                srcv = sbuf[b][pl.ds(t * 16, 16)]
                dstv = dbuf[b][pl.ds(t * 16, 16)]
                match = ((dstv >> 5) & 31) == w_vec
                loc = ((dstv >> 10) << 5) | (dstv & 31)
                plsc.store_compressed(stag_s.at[pl.ds(n, 16)], srcv,
                                      mask=match)
                plsc.store_compressed(stag_l.at[pl.ds(n, 16)], loc,
                                      mask=match)
                plsc.addupdate_scatter(degv, [dstv], ones16, mask=match)
                n = n + jnp.sum(match.astype(jnp.int32))

            def _flush(args):
                n, flushed = args
                pltpu.sync_copy(stag_s.at[pl.ds(0, FLUSH)],
                                slist_hbm.at[w, pl.ds(flushed * FLUSH, FLUSH)])
                pltpu.sync_copy(stag_l.at[pl.ds(0, FLUSH)],
                                llist_hbm.at[w, pl.ds(flushed * FLUSH, FLUSH)])
                rem = n - FLUSH

                def _shift(i, carry2):
                    sv = stag_s[pl.ds(FLUSH + i * 16, 16)]
                    lv = stag_l[pl.ds(FLUSH + i * 16, 16)]
                    stag_s[pl.ds(i * 16, 16)] = sv
                    stag_l[pl.ds(i * 16, 16)] = lv
                    return carry2

                lax.fori_loop(0, (rem + 15) >> 4, _shift, 0)
                return rem, flushed + 1

            carry = lax.cond(n >= FLUSH, _flush, lambda args: args,
                             (n, flushed))
        return carry

    n, flushed = lax.fori_loop(0, N_BLKS // 2, _pairblk,
                               (jnp.int32(0), jnp.int32(0)))

    zero16 = jnp.zeros((16,), jnp.int32)
    trash16 = jnp.full((16,), TRASH, jnp.int32)
    for t in range(8):
        stag_s[pl.ds(n + t * 16, 16)] = zero16
        stag_l[pl.ds(n + t * 16, 16)] = trash16
    n_pad = ((n + 127) >> 7) << 7
    pltpu.sync_copy(stag_s.at[pl.ds(0, FINAL_FLUSH)],
                    slist_hbm.at[w, pl.ds(flushed * FLUSH, FINAL_FLUSH)])
    pltpu.sync_copy(stag_l.at[pl.ds(0, FINAL_FLUSH)],
                    llist_hbm.at[w, pl.ds(flushed * FLUSH, FINAL_FLUSH)])
    cbuf[...] = jnp.zeros((16,), jnp.int32) + (flushed * FLUSH + n_pad)
    pltpu.sync_copy(cbuf, cnt_hbm.at[w])
    pltpu.sync_copy(degv, deg_hbm.at[w])


@functools.cache
def _get_scatter_kernel():
    return pl.kernel(
        _scatter_body,
        out_type=jax.ShapeDtypeStruct((N_PAD, D), jnp.float32),
        mesh=_mesh(),
        scratch_types=[
            pltpu.VMEM((ACC_ROWS, D), jnp.float32),
            pltpu.VMEM((GRP,), jnp.int32),
            pltpu.VMEM((GRP,), jnp.int32),
            pltpu.VMEM((GRP,), jnp.int32),
            pltpu.VMEM((GRP,), jnp.int32),
            pltpu.VMEM((GRP, D), jnp.float32),
            pltpu.VMEM((GRP, D), jnp.float32),
            pltpu.VMEM((16,), jnp.int32),
            pltpu.SemaphoreType.DMA,
            pltpu.SemaphoreType.DMA,
            pltpu.SemaphoreType.DMA,
            pltpu.SemaphoreType.DMA,
        ],
        compiler_params=pltpu.CompilerParams(needs_layout_passes=False),
    )


def _scatter_body(g_hbm, slist_hbm, llist_hbm, cnt_hbm, zeros_hbm, out_hbm,
                  acc, sidx0, sidx1, locv0, locv1, rows0, rows1, cbuf,
                  sem0, sem1, semi0, semi1):
    w = _wid()
    pltpu.sync_copy(cnt_hbm.at[w], cbuf)
    cnt = jnp.max(cbuf[pl.ds(0, 16)])
    n_groups = cnt >> 6

    pltpu.sync_copy(zeros_hbm, acc)

    sidx = (sidx0, sidx1)
    locv = (locv0, locv1)
    rows = (rows0, rows1)
    sem = (sem0, sem1)
    semi = (semi0, semi1)
    iota16 = lax.iota(jnp.int32, 16)
    col = [iota16 + t * 16 for t in range(16)]

    pltpu.sync_copy(slist_hbm.at[w, pl.ds(0, GRP)], sidx0)
    pltpu.sync_copy(llist_hbm.at[w, pl.ds(0, GRP)], locv0)
    pltpu.async_copy(g_hbm.at[sidx0], rows0, sem0)

    @pl.when(1 < n_groups)
    def _prime_idx():
        pltpu.async_copy(slist_hbm.at[w, pl.ds(GRP, GRP)], sidx1, semi1)
        pltpu.async_copy(llist_hbm.at[w, pl.ds(GRP, GRP)], locv1, semi1)

    def _pair(p, carry):
        for b in range(2):
            g = p * 2 + b
            nb = 1 - b
            pltpu.make_async_copy(g_hbm.at[sidx[b]], rows[b], sem[b]).wait()

            @pl.when(g + 1 < n_groups)
            def _launch_next_gather():
                pltpu.make_async_copy(slist_hbm.at[w, pl.ds(0, GRP)],
                                      sidx[nb], semi[nb]).wait()
                pltpu.make_async_copy(llist_hbm.at[w, pl.ds(0, GRP)],
                                      locv[nb], semi[nb]).wait()
                pltpu.async_copy(g_hbm.at[sidx[nb]], rows[nb], sem[nb])

            rbuf = rows[b]
            lbuf = locv[b]

            @pl.when(g < n_groups)
            def _accumulate():
                def _row(r):
                    lr = plsc.load_gather(lbuf, [jnp.zeros((16,), jnp.int32) + r])
                    for t in range(16):
                        vals = rbuf[r, pl.ds(t * 16, 16)]
                        plsc.addupdate_scatter(acc, [lr, col[t]], vals)

                plsc.parallel_loop(0, GRP, unroll=2)(_row)

            @pl.when(g + 2 < n_groups)
            def _prefetch_idx():
                off = (g + 2) * GRP
                pltpu.async_copy(slist_hbm.at[w, pl.ds(off, GRP)], sidx[b],
                                 semi[b])
                pltpu.async_copy(llist_hbm.at[w, pl.ds(off, GRP)], locv[b],
                                 semi[b])

        return carry

    lax.fori_loop(0, (n_groups + 1) >> 1, _pair, 0)

    for blk in range(10):
        pltpu.sync_copy(acc.at[pl.ds(blk * 32, 32)],
                        out_hbm.at[pl.ds(blk * 1024 + w * 32, 32)])


def _dinv_block(degp_blk):
    ones = jnp.ones((NW, 256), jnp.float32)
    degsum = lax.dot_general(degp_blk, ones, (((0,), (0,)), ((), ())),
                             preferred_element_type=jnp.float32)
    return lax.rsqrt(degsum + 1.0)


def _mm1_body(x_blk, w1, degp_blk, out_blk):
    dinv = _dinv_block(degp_blk[...])
    h = lax.dot_general(x_blk[...], w1[...], (((1,), (0,)), ((), ())),
                        preferred_element_type=jnp.float32,
                        precision=lax.Precision.HIGHEST)
    out_blk[...] = dinv * h


def _combine_mm_body(s_blk, g_blk, degp_blk, b_blk, w_blk, out_blk):
    dinv = _dinv_block(degp_blk[...])
    u = jnp.maximum(dinv * (s_blk[...] + g_blk[...]) + b_blk[...], 0.0)
    h = lax.dot_general(u, w_blk[...], (((1,), (0,)), ((), ())),
                        preferred_element_type=jnp.float32,
                        precision=lax.Precision.HIGHEST)
    out_blk[...] = dinv * h


def _final_body(s_blk, g_blk, degp_blk, b3, wo1, bo1, wo2, bo2, out_blk):
    dinv = _dinv_block(degp_blk[...])
    u = jnp.maximum(dinv * (s_blk[...] + g_blk[...]) + b3[...], 0.0)
    t = lax.dot_general(u, wo1[...], (((1,), (0,)), ((), ())),
                        preferred_element_type=jnp.float32,
                        precision=lax.Precision.HIGHEST) + bo1[...]
    out_blk[...] = lax.dot_general(t, wo2[...], (((1,), (0,)), ((), ())),
                                   preferred_element_type=jnp.float32,
                                   precision=lax.Precision.HIGHEST) + bo2[...]


def _row_spec(w):
    return pl.BlockSpec((256, w), lambda i: (i, 0))


def _full_spec(shape):
    return pl.BlockSpec(shape, lambda i: (0,) * len(shape))


def _mm1(x, w1, degp):
    return pl.pallas_call(
        _mm1_body,
        grid=(N_PAD // 256,),
        in_specs=[_row_spec(128), _full_spec((128, 256)),
                  pl.BlockSpec((NW, 256), lambda i: (0, i))],
        out_specs=_row_spec(256),
        out_shape=jax.ShapeDtypeStruct((N_PAD, 256), jnp.float32),
    )(x, w1, degp)


def _combine_mm(sagg, g, degp, b, w):
    return pl.pallas_call(
        _combine_mm_body,
        grid=(N_PAD // 256,),
        in_specs=[_row_spec(256), _row_spec(256),
                  pl.BlockSpec((NW, 256), lambda i: (0, i)),
                  _full_spec((1, 256)), _full_spec((256, 256))],
        out_specs=_row_spec(256),
        out_shape=jax.ShapeDtypeStruct((N_PAD, 256), jnp.float32),
    )(sagg, g, degp, b, w)


def _final(sagg, g, degp, b3, wo1, bo1, wo2, bo2):
    return pl.pallas_call(
        _final_body,
        grid=(N_PAD // 256,),
        in_specs=[_row_spec(256), _row_spec(256),
                  pl.BlockSpec((NW, 256), lambda i: (0, i)),
                  _full_spec((1, 256)), _full_spec((256, 256)),
                  _full_spec((1, 256)), _full_spec((256, 128)),
                  _full_spec((1, 128))],
        out_specs=_row_spec(128),
        out_shape=jax.ShapeDtypeStruct((N_PAD, 128), jnp.float32),
    )(sagg, g, degp, b3, wo1, bo1, wo2, bo2)


def kernel(x, edge_index, W1, b1, W2, b2, W3, b3, Wo1, bo1, Wo2, bo2):
    src = edge_index[0].astype(jnp.int32)
    dst = edge_index[1].astype(jnp.int32)
    xp = jnp.zeros((N_PAD, 128), jnp.float32).at[:N].set(x)
    zeros_acc = jnp.zeros((ACC_ROWS, D), jnp.float32)

    slist, llist, cnts, degp = _get_route_kernel()(src, dst)
    scatter = _get_scatter_kernel()

    g1 = _mm1(xp, W1, degp)
    s1 = scatter(g1, slist, llist, cnts, zeros_acc)
    g2 = _combine_mm(s1, g1, degp, b1.reshape(1, 256), W2)
    s2 = scatter(g2, slist, llist, cnts, zeros_acc)
    g3 = _combine_mm(s2, g2, degp, b2.reshape(1, 256), W3)
    s3 = scatter(g3, slist, llist, cnts, zeros_acc)
    y = _final(s3, g3, degp, b3.reshape(1, 256), Wo1, bo1.reshape(1, 256),
               Wo2, bo2.reshape(1, 128))
    return y[:N]


# speedup vs baseline: 1.0677x; 1.0677x over previous
import functools

import jax
import jax.numpy as jnp
from jax import lax
from jax.experimental import pallas as pl
from jax.experimental.pallas import tpu as pltpu
from jax.experimental.pallas import tpu_sc as plsc

N = 10000
N_PAD = 10240
E = 320000
NC, NS = 2, 16
NW = NC * NS
E_PER_W = E // NW
D = 256

BLK = 640
DEG_BLK = 400
N_BLKS = E // BLK
STAG = 1792
FLUSH = 1024
FINAL_FLUSH = 1152
LIST_CAP = E + 2048
GRP = 64
ROWS_PER_TILE = 320
TRASH = ROWS_PER_TILE
ACC_ROWS = ROWS_PER_TILE + 8


def _mesh():
    return plsc.VectorSubcoreMesh(core_axis_name="c", subcore_axis_name="s",
                                  num_cores=NC, num_subcores=NS)


def _wid():
    return lax.axis_index("s") * NC + lax.axis_index("c")


@functools.cache
def _get_deg_kernel():
    return pl.kernel(
        _deg_body,
        out_type=jax.ShapeDtypeStruct((NW, N_PAD), jnp.float32),
        mesh=_mesh(),
        scratch_types=[
            pltpu.VMEM((N_PAD,), jnp.float32),
            pltpu.VMEM((DEG_BLK,), jnp.int32),
        ],
        compiler_params=pltpu.CompilerParams(needs_layout_passes=False),
    )


def _deg_body(dst_hbm, out_hbm, cnt_v, idx_v):
    base = _wid() * E_PER_W

    def _zero(i, carry):
        cnt_v[pl.ds(i * 16, 16)] = jnp.zeros((16,), jnp.float32)
        return carry

    lax.fori_loop(0, N_PAD // 16, _zero, 0)

    ones16 = jnp.ones((16,), jnp.float32)

    def _blk(j, carry):
        pltpu.sync_copy(dst_hbm.at[pl.ds(base + j * DEG_BLK, DEG_BLK)], idx_v)
        for t in range(DEG_BLK // 16):
            idx = idx_v[pl.ds(t * 16, 16)]
            plsc.addupdate_scatter(cnt_v, [idx], ones16)
        return carry

    lax.fori_loop(0, E_PER_W // DEG_BLK, _blk, 0)
    pltpu.sync_copy(cnt_v, out_hbm.at[_wid()])


@functools.cache
def _get_route_kernel():
    return pl.kernel(
        _route_body,
        out_type=(
            jax.ShapeDtypeStruct((NW, LIST_CAP), jnp.int32),
            jax.ShapeDtypeStruct((NW, LIST_CAP), jnp.int32),
            jax.ShapeDtypeStruct((NW, 16), jnp.int32),
        ),
        mesh=_mesh(),
        scratch_types=[
            pltpu.VMEM((BLK,), jnp.int32),
            pltpu.VMEM((BLK,), jnp.int32),
            pltpu.VMEM((BLK,), jnp.int32),
            pltpu.VMEM((BLK,), jnp.int32),
            pltpu.VMEM((STAG,), jnp.int32),
            pltpu.VMEM((STAG,), jnp.int32),
            pltpu.VMEM((16,), jnp.int32),
            pltpu.SemaphoreType.DMA,
            pltpu.SemaphoreType.DMA,
        ],
        compiler_params=pltpu.CompilerParams(needs_layout_passes=False),
    )


def _route_body(src_hbm, dst_hbm, slist_hbm, llist_hbm, cnt_hbm,
                sbuf0, dbuf0, sbuf1, dbuf1, stag_s, stag_l, cbuf,
                semb0, semb1):
    w = _wid()
    w_vec = jnp.full((16,), 0, jnp.int32) + w
    sbuf = (sbuf0, sbuf1)
    dbuf = (dbuf0, dbuf1)
    semb = (semb0, semb1)

    pltpu.async_copy(src_hbm.at[pl.ds(0, BLK)], sbuf0, semb0)
    pltpu.async_copy(dst_hbm.at[pl.ds(0, BLK)], dbuf0, semb0)

    def _pairblk(p, carry):
        for b in range(2):
            j = p * 2 + b
            pltpu.make_async_copy(src_hbm.at[pl.ds(0, BLK)], sbuf[b],
                                  semb[b]).wait()
            pltpu.make_async_copy(dst_hbm.at[pl.ds(0, BLK)], dbuf[b],
                                  semb[b]).wait()

            @pl.when(j + 1 < N_BLKS)
            def _prefetch():
                nb = 1 - b
                off = (j + 1) * BLK
                pltpu.async_copy(src_hbm.at[pl.ds(off, BLK)], sbuf[nb],
                                 semb[nb])
                pltpu.async_copy(dst_hbm.at[pl.ds(off, BLK)], dbuf[nb],
                                 semb[nb])

            n, flushed = carry
            for t in range(BLK // 16):
                srcv = sbuf[b][pl.ds(t * 16, 16)]
                dstv = dbuf[b][pl.ds(t * 16, 16)]
                match = ((dstv >> 5) & 31) == w_vec
                loc = ((dstv >> 10) << 5) | (dstv & 31)
                plsc.store_compressed(stag_s.at[pl.ds(n, 16)], srcv,
                                      mask=match)
                plsc.store_compressed(stag_l.at[pl.ds(n, 16)], loc,
                                      mask=match)
                n = n + jnp.sum(match.astype(jnp.int32))

            def _flush(args):
                n, flushed = args
                pltpu.sync_copy(stag_s.at[pl.ds(0, FLUSH)],
                                slist_hbm.at[w, pl.ds(flushed * FLUSH, FLUSH)])
                pltpu.sync_copy(stag_l.at[pl.ds(0, FLUSH)],
                                llist_hbm.at[w, pl.ds(flushed * FLUSH, FLUSH)])
                rem = n - FLUSH

                def _shift(i, carry2):
                    sv = stag_s[pl.ds(FLUSH + i * 16, 16)]
                    lv = stag_l[pl.ds(FLUSH + i * 16, 16)]
                    stag_s[pl.ds(i * 16, 16)] = sv
                    stag_l[pl.ds(i * 16, 16)] = lv
                    return carry2

                lax.fori_loop(0, (rem + 15) >> 4, _shift, 0)
                return rem, flushed + 1

            carry = lax.cond(n >= FLUSH, _flush, lambda args: args,
                             (n, flushed))
        return carry

    n, flushed = lax.fori_loop(0, N_BLKS // 2, _pairblk,
                               (jnp.int32(0), jnp.int32(0)))

    zero16 = jnp.zeros((16,), jnp.int32)
    trash16 = jnp.full((16,), TRASH, jnp.int32)
    for t in range(8):
        stag_s[pl.ds(n + t * 16, 16)] = zero16
        stag_l[pl.ds(n + t * 16, 16)] = trash16
    n_pad = ((n + 127) >> 7) << 7
    pltpu.sync_copy(stag_s.at[pl.ds(0, FINAL_FLUSH)],
                    slist_hbm.at[w, pl.ds(flushed * FLUSH, FINAL_FLUSH)])
    pltpu.sync_copy(stag_l.at[pl.ds(0, FINAL_FLUSH)],
                    llist_hbm.at[w, pl.ds(flushed * FLUSH, FINAL_FLUSH)])
    cbuf[...] = jnp.zeros((16,), jnp.int32) + (flushed * FLUSH + n_pad)
    pltpu.sync_copy(cbuf, cnt_hbm.at[w])


@functools.cache
def _get_scatter_kernel():
    return pl.kernel(
        _scatter_body,
        out_type=jax.ShapeDtypeStruct((N_PAD, D), jnp.float32),
        mesh=_mesh(),
        scratch_types=[
            pltpu.VMEM((ACC_ROWS, D), jnp.float32),
            pltpu.VMEM((GRP,), jnp.int32),
            pltpu.VMEM((GRP,), jnp.int32),
            pltpu.VMEM((GRP,), jnp.int32),
            pltpu.VMEM((GRP,), jnp.int32),
            pltpu.VMEM((GRP, D), jnp.float32),
            pltpu.VMEM((GRP, D), jnp.float32),
            pltpu.VMEM((16,), jnp.int32),
            pltpu.SemaphoreType.DMA,
            pltpu.SemaphoreType.DMA,
            pltpu.SemaphoreType.DMA,
            pltpu.SemaphoreType.DMA,
        ],
        compiler_params=pltpu.CompilerParams(needs_layout_passes=False),
    )


def _scatter_body(g_hbm, slist_hbm, llist_hbm, cnt_hbm, zeros_hbm, out_hbm,
                  acc, sidx0, sidx1, locv0, locv1, rows0, rows1, cbuf,
                  sem0, sem1, semi0, semi1):
    w = _wid()
    pltpu.sync_copy(cnt_hbm.at[w], cbuf)
    cnt = jnp.max(cbuf[pl.ds(0, 16)])
    n_groups = cnt >> 6

    pltpu.sync_copy(zeros_hbm, acc)

    sidx = (sidx0, sidx1)
    locv = (locv0, locv1)
    rows = (rows0, rows1)
    sem = (sem0, sem1)
    semi = (semi0, semi1)
    iota16 = lax.iota(jnp.int32, 16)
    col = [iota16 + t * 16 for t in range(16)]

    pltpu.sync_copy(slist_hbm.at[w, pl.ds(0, GRP)], sidx0)
    pltpu.sync_copy(llist_hbm.at[w, pl.ds(0, GRP)], locv0)
    pltpu.async_copy(g_hbm.at[sidx0], rows0, sem0)

    @pl.when(1 < n_groups)
    def _prime_idx():
        pltpu.async_copy(slist_hbm.at[w, pl.ds(GRP, GRP)], sidx1, semi1)
        pltpu.async_copy(llist_hbm.at[w, pl.ds(GRP, GRP)], locv1, semi1)

    def _pair(p, carry):
        for b in range(2):
            g = p * 2 + b
            nb = 1 - b
            pltpu.make_async_copy(g_hbm.at[sidx[b]], rows[b], sem[b]).wait()

            @pl.when(g + 1 < n_groups)
            def _launch_next_gather():
                pltpu.make_async_copy(slist_hbm.at[w, pl.ds(0, GRP)],
                                      sidx[nb], semi[nb]).wait()
                pltpu.make_async_copy(llist_hbm.at[w, pl.ds(0, GRP)],
                                      locv[nb], semi[nb]).wait()
                pltpu.async_copy(g_hbm.at[sidx[nb]], rows[nb], sem[nb])

            rbuf = rows[b]
            lbuf = locv[b]

            @pl.when(g < n_groups)
            def _accumulate():
                def _row(r):
                    lr = plsc.load_gather(lbuf, [jnp.zeros((16,), jnp.int32) + r])
                    for t in range(16):
                        vals = rbuf[r, pl.ds(t * 16, 16)]
                        plsc.addupdate_scatter(acc, [lr, col[t]], vals)

                plsc.parallel_loop(0, GRP, unroll=2)(_row)

            @pl.when(g + 2 < n_groups)
            def _prefetch_idx():
                off = (g + 2) * GRP
                pltpu.async_copy(slist_hbm.at[w, pl.ds(off, GRP)], sidx[b],
                                 semi[b])
                pltpu.async_copy(llist_hbm.at[w, pl.ds(off, GRP)], locv[b],
                                 semi[b])

        return carry

    lax.fori_loop(0, (n_groups + 1) >> 1, _pair, 0)

    for blk in range(10):
        pltpu.sync_copy(acc.at[pl.ds(blk * 32, 32)],
                        out_hbm.at[pl.ds(blk * 1024 + w * 32, 32)])


def _dinv_block(degp_blk):
    ones = jnp.ones((NW, 256), jnp.float32)
    degsum = lax.dot_general(degp_blk, ones, (((0,), (0,)), ((), ())),
                             preferred_element_type=jnp.float32)
    return lax.rsqrt(degsum + 1.0)


def _mm1_body(x_blk, w1, degp_blk, out_blk):
    dinv = _dinv_block(degp_blk[...])
    h = lax.dot_general(x_blk[...], w1[...], (((1,), (0,)), ((), ())),
                        preferred_element_type=jnp.float32,
                        precision=lax.Precision.HIGHEST)
    out_blk[...] = dinv * h


def _combine_mm_body(s_blk, g_blk, degp_blk, b_blk, w_blk, out_blk):
    dinv = _dinv_block(degp_blk[...])
    u = jnp.maximum(dinv * (s_blk[...] + g_blk[...]) + b_blk[...], 0.0)
    h = lax.dot_general(u, w_blk[...], (((1,), (0,)), ((), ())),
                        preferred_element_type=jnp.float32,
                        precision=lax.Precision.HIGHEST)
    out_blk[...] = dinv * h


def _final_body(s_blk, g_blk, degp_blk, b3, wo1, bo1, wo2, bo2, out_blk):
    dinv = _dinv_block(degp_blk[...])
    u = jnp.maximum(dinv * (s_blk[...] + g_blk[...]) + b3[...], 0.0)
    t = lax.dot_general(u, wo1[...], (((1,), (0,)), ((), ())),
                        preferred_element_type=jnp.float32,
                        precision=lax.Precision.HIGHEST) + bo1[...]
    out_blk[...] = lax.dot_general(t, wo2[...], (((1,), (0,)), ((), ())),
                                   preferred_element_type=jnp.float32,
                                   precision=lax.Precision.HIGHEST) + bo2[...]


def _row_spec(w):
    return pl.BlockSpec((256, w), lambda i: (i, 0))


def _full_spec(shape):
    return pl.BlockSpec(shape, lambda i: (0,) * len(shape))


def _mm1(x, w1, degp):
    return pl.pallas_call(
        _mm1_body,
        grid=(N_PAD // 256,),
        in_specs=[_row_spec(128), _full_spec((128, 256)),
                  pl.BlockSpec((NW, 256), lambda i: (0, i))],
        out_specs=_row_spec(256),
        out_shape=jax.ShapeDtypeStruct((N_PAD, 256), jnp.float32),
    )(x, w1, degp)


def _combine_mm(sagg, g, degp, b, w):
    return pl.pallas_call(
        _combine_mm_body,
        grid=(N_PAD // 256,),
        in_specs=[_row_spec(256), _row_spec(256),
                  pl.BlockSpec((NW, 256), lambda i: (0, i)),
                  _full_spec((1, 256)), _full_spec((256, 256))],
        out_specs=_row_spec(256),
        out_shape=jax.ShapeDtypeStruct((N_PAD, 256), jnp.float32),
    )(sagg, g, degp, b, w)


def _final(sagg, g, degp, b3, wo1, bo1, wo2, bo2):
    return pl.pallas_call(
        _final_body,
        grid=(N_PAD // 256,),
        in_specs=[_row_spec(256), _row_spec(256),
                  pl.BlockSpec((NW, 256), lambda i: (0, i)),
                  _full_spec((1, 256)), _full_spec((256, 256)),
                  _full_spec((1, 256)), _full_spec((256, 128)),
                  _full_spec((1, 128))],
        out_specs=_row_spec(128),
        out_shape=jax.ShapeDtypeStruct((N_PAD, 128), jnp.float32),
    )(sagg, g, degp, b3, wo1, bo1, wo2, bo2)


def kernel(x, edge_index, W1, b1, W2, b2, W3, b3, Wo1, bo1, Wo2, bo2):
    src = edge_index[0].astype(jnp.int32)
    dst = edge_index[1].astype(jnp.int32)
    xp = jnp.zeros((N_PAD, 128), jnp.float32).at[:N].set(x)
    zeros_acc = jnp.zeros((ACC_ROWS, D), jnp.float32)

    degp = _get_deg_kernel()(dst)
    slist, llist, cnts = _get_route_kernel()(src, dst)
    scatter = _get_scatter_kernel()

    g1 = _mm1(xp, W1, degp)
    s1 = scatter(g1, slist, llist, cnts, zeros_acc)
    g2 = _combine_mm(s1, g1, degp, b1.reshape(1, 256), W2)
    s2 = scatter(g2, slist, llist, cnts, zeros_acc)
    g3 = _combine_mm(s2, g2, degp, b2.reshape(1, 256), W3)
    s3 = scatter(g3, slist, llist, cnts, zeros_acc)
    y = _final(s3, g3, degp, b3.reshape(1, 256), Wo1, bo1.reshape(1, 256),
               Wo2, bo2.reshape(1, 128))
    return y[:N]
